# Initial kernel scaffold; baseline (speedup 1.0000x reference)
import jax
import jax.numpy as jnp
from jax.experimental import pallas as pl


def kernel(x, noise, Wr, br, Wn, bn, W1, b1, W2, b2):
    raise NotImplementedError("write your pallas kernel here")


# speedup vs baseline: 3.3157x; 3.3157x over previous
import functools

import jax
import jax.numpy as jnp
from jax import lax
from jax.experimental import pallas as pl
from jax.experimental.pallas import tpu as pltpu
from jax.experimental.pallas import tpu_sc as plsc

TOP_K = 2
NC, NS, L = 2, 16, 16
NW = NC * NS


def _router_body(x_ref, wrt_ref, br_ref, wnt_ref, bn_ref, noiset_ref,
                 e1_ref, e2_ref, g1_ref, g2_ref):
    x = x_ref[...]
    dn = (((1,), (1,)), ((), ()))
    lg = lax.dot_general(wrt_ref[...], x, dn,
                         preferred_element_type=jnp.float32) + br_ref[...]
    nl = lax.dot_general(wnt_ref[...], x, dn,
                         preferred_element_type=jnp.float32) + bn_ref[...]
    sp = jnp.maximum(nl, 0.0) + jnp.log(1.0 + jnp.exp(-jnp.abs(nl)))
    noisy = lg + noiset_ref[...] * sp

    E = noisy.shape[0]
    iota = lax.broadcasted_iota(jnp.int32, noisy.shape, 0)
    m1 = jnp.max(noisy, axis=0)
    e1 = jnp.min(jnp.where(noisy == m1[None, :], iota, E), axis=0)
    masked = jnp.where(iota == e1[None, :], -jnp.inf, noisy)
    m2 = jnp.max(masked, axis=0)
    e2 = jnp.min(jnp.where(masked == m2[None, :], iota, E), axis=0)
    z = jnp.exp(m2 - m1)
    denom = 1.0 + z
    e1_ref[...] = e1[None, :]
    e2_ref[...] = e2[None, :]
    g1_ref[...] = (1.0 / denom)[None, :]
    g2_ref[...] = (z / denom)[None, :]


def _router_call(xf, WrT, brc, WnT, bnc, noiseT):
    N = xf.shape[0]
    return pl.pallas_call(
        _router_body,
        out_shape=(
            jax.ShapeDtypeStruct((1, N), jnp.int32),
            jax.ShapeDtypeStruct((1, N), jnp.int32),
            jax.ShapeDtypeStruct((1, N), jnp.float32),
            jax.ShapeDtypeStruct((1, N), jnp.float32),
        ),
    )(xf, WrT, brc, WnT, bnc, noiseT)


def _make_compact_kernel(N, E, CAP):
    NCHUNK = N // L
    mesh = plsc.VectorSubcoreMesh(core_axis_name="c", subcore_axis_name="s")

    @functools.partial(
        pl.kernel,
        out_type=(
            jax.ShapeDtypeStruct((E, CAP), jnp.int32),
            jax.ShapeDtypeStruct((E, CAP), jnp.float32),
            jax.ShapeDtypeStruct((E, N), jnp.int32),
            jax.ShapeDtypeStruct((E, L), jnp.int32),
        ),
        mesh=mesh,
        compiler_params=pltpu.CompilerParams(needs_layout_passes=False),
        scratch_types=[
            pltpu.VMEM((N,), jnp.int32),
            pltpu.VMEM((N,), jnp.int32),
            pltpu.VMEM((N,), jnp.float32),
            pltpu.VMEM((N,), jnp.float32),
            pltpu.VMEM((N + L,), jnp.int32),
            pltpu.VMEM((N + L,), jnp.float32),
            pltpu.VMEM((N,), jnp.int32),
            pltpu.VMEM((L,), jnp.int32),
        ],
    )
    def compact(e1_hbm, e2_hbm, g1_hbm, g2_hbm,
                tok_hbm, gate_hbm, slot_hbm, cnt_hbm,
                e1b, e2b, g1b, g2b, tokb, gateb, slotb, cntb):
        wid = lax.axis_index("s") * NC + lax.axis_index("c")

        @pl.when(wid < E)
        def _():
            eid = wid
            pltpu.sync_copy(e1_hbm.at[0], e1b)
            pltpu.sync_copy(e2_hbm.at[0], e2b)
            pltpu.sync_copy(g1_hbm.at[0], g1b)
            pltpu.sync_copy(g2_hbm.at[0], g2b)

            zi = jnp.zeros((L,), jnp.int32)
            zf = jnp.zeros((L,), jnp.float32)

            def _zero(i, carry):
                tokb[pl.ds(i * L, L)] = zi
                gateb[pl.ds(i * L, L)] = zf
                return carry

            lax.fori_loop(0, CAP // L, _zero, 0)

            iota = lax.iota(jnp.int32, L)

            def _scan(c, off):
                ve1 = e1b[pl.ds(c * L, L)]
                ve2 = e2b[pl.ds(c * L, L)]
                m1 = ve1 == eid
                m2 = ve2 == eid
                mask = jnp.logical_or(m1, m2)
                mi = mask.astype(jnp.int32)
                inc = plsc.cumsum(mi)
                slotv = off + (inc - mi)
                slotb[pl.ds(c * L, L)] = slotv
                g = jnp.where(m1, g1b[pl.ds(c * L, L)],
                              jnp.where(m2, g2b[pl.ds(c * L, L)], 0.0))
                tokv = c * L + iota
                plsc.store_compressed(tokb.at[pl.ds(off, L)], tokv, mask=mask)
                plsc.store_compressed(gateb.at[pl.ds(off, L)], g, mask=mask)
                return off + jnp.sum(mi)

            cnt = lax.fori_loop(0, NCHUNK, _scan, jnp.int32(0))

            pltpu.sync_copy(tokb.at[pl.ds(0, CAP)], tok_hbm.at[eid])
            pltpu.sync_copy(gateb.at[pl.ds(0, CAP)], gate_hbm.at[eid])
            pltpu.sync_copy(slotb, slot_hbm.at[eid])
            cntb[pl.ds(0, L)] = jnp.full((L,), cnt, jnp.int32)
            pltpu.sync_copy(cntb, cnt_hbm.at[eid])

    return compact


def _make_dispatch_kernel(N, C, E, CAP):
    TPW = N // NW
    NCH = TPW // L
    RPW = (E * CAP) // NW
    GC = 64
    TPE = NW // E
    mesh = plsc.VectorSubcoreMesh(core_axis_name="c", subcore_axis_name="s")

    @functools.partial(
        pl.kernel,
        out_type=(
            jax.ShapeDtypeStruct((E * CAP, C), jnp.float32),
            jax.ShapeDtypeStruct((N,), jnp.int32),
            jax.ShapeDtypeStruct((N,), jnp.int32),
        ),
        mesh=mesh,
        scratch_types=[
            pltpu.VMEM((E, TPW), jnp.int32),
            pltpu.VMEM((E, L), jnp.int32),
            pltpu.VMEM((TPW,), jnp.int32),
            pltpu.VMEM((TPW,), jnp.int32),
            pltpu.VMEM((TPW,), jnp.int32),
            pltpu.VMEM((TPW,), jnp.int32),
            pltpu.VMEM((RPW,), jnp.int32),
            pltpu.VMEM((GC, C), jnp.float32),
            pltpu.SemaphoreType.DMA,
        ],
    )
    def dispatch(x_hbm, e1_hbm, e2_hbm, tok_hbm, slot_hbm, cnt_hbm,
                 xe_hbm, p1_hbm, p2_hbm,
                 slotm, cnts, e1b, e2b, p1b, p2b, idxb, rowb, sem):
        wid = lax.axis_index("s") * NC + lax.axis_index("c")
        t0 = wid * TPW

        pltpu.sync_copy(slot_hbm.at[:, pl.ds(t0, TPW)], slotm)
        pltpu.sync_copy(cnt_hbm, cnts)
        pltpu.sync_copy(e1_hbm.at[0, pl.ds(t0, TPW)], e1b)
        pltpu.sync_copy(e2_hbm.at[0, pl.ds(t0, TPW)], e2b)

        ez = jnp.int32(-1)
        for e in range(E):
            tot = cnts[e][0]
            take = jnp.logical_and(tot < CAP, ez < 0)
            ez = jnp.where(take, jnp.int32(e), ez)
        zero_flat = jnp.where(ez >= 0, ez * CAP + (CAP - 1), 0)

        for c in range(NCH):
            ve1 = e1b[pl.ds(c * L, L)]
            ve2 = e2b[pl.ds(c * L, L)]
            s1 = jnp.zeros((L,), jnp.int32)
            s2 = jnp.zeros((L,), jnp.int32)
            for e in range(E):
                row = slotm[e, pl.ds(c * L, L)]
                s1 = jnp.where(ve1 == e, row, s1)
                s2 = jnp.where(ve2 == e, row, s2)
            p1v = jnp.where(s1 < CAP, ve1 * CAP + s1, zero_flat)
            p2v = jnp.where(s2 < CAP, ve2 * CAP + s2, zero_flat)
            p1b[pl.ds(c * L, L)] = p1v
            p2b[pl.ds(c * L, L)] = p2v

        pltpu.sync_copy(p1b, p1_hbm.at[pl.ds(t0, TPW)])
        pltpu.sync_copy(p2b, p2_hbm.at[pl.ds(t0, TPW)])

        eid = wid // TPE
        s0 = (wid % TPE) * RPW
        pltpu.sync_copy(tok_hbm.at[eid, pl.ds(s0, RPW)], idxb)
        r0 = wid * RPW
        for k in range(RPW // GC):
            pltpu.async_copy(x_hbm.at[idxb.at[pl.ds(k * GC, GC)]], rowb,
                             sem).wait()
            pltpu.sync_copy(rowb, xe_hbm.at[pl.ds(r0 + k * GC, GC)])

    return dispatch


def _ffn_body(xe_ref, w1_ref, b1_ref, w2_ref, b2_ref, gate_ref, y_ref,
              acc_ref):
    f = pl.program_id(1)
    nf = pl.num_programs(1)
    h = jnp.dot(xe_ref[...], w1_ref[0], preferred_element_type=jnp.float32)
    h = jnp.maximum(h + b1_ref[0], 0.0)
    part = jnp.dot(h, w2_ref[0], preferred_element_type=jnp.float32)

    @pl.when(f == 0)
    def _():
        acc_ref[...] = jnp.zeros_like(acc_ref)

    acc_ref[...] += part

    @pl.when(f == nf - 1)
    def _():
        y_ref[...] = (acc_ref[...] + b2_ref[0]) * gate_ref[...]


def _ffn_call(xe, W1, b1, W2, b2, gate_map, CAP, FB):
    E, C, F = W1.shape
    grid = (E, F // FB)
    return pl.pallas_call(
        _ffn_body,
        grid=grid,
        in_specs=[
            pl.BlockSpec((CAP, C), lambda e, f: (e, 0)),
            pl.BlockSpec((1, C, FB), lambda e, f: (e, 0, f)),
            pl.BlockSpec((1, 1, FB), lambda e, f: (e, 0, f)),
            pl.BlockSpec((1, FB, C), lambda e, f: (e, f, 0)),
            pl.BlockSpec((1, 1, C), lambda e, f: (e, 0, 0)),
            pl.BlockSpec((CAP, 1), lambda e, f: (e, 0)),
        ],
        out_specs=pl.BlockSpec((CAP, C), lambda e, f: (e, 0)),
        out_shape=jax.ShapeDtypeStruct((E * CAP, C), jnp.float32),
        scratch_shapes=[pltpu.VMEM((CAP, C), jnp.float32)],
        compiler_params=pltpu.CompilerParams(
            dimension_semantics=("arbitrary", "arbitrary")),
    )(xe, W1, b1.reshape(E, 1, F), W2, b2.reshape(E, 1, C),
      gate_map.reshape(E * CAP, 1))


def _make_combine_kernel(N, C, YROWS):
    TPW = N // NW
    GC = 64
    CV = C // L
    mesh = plsc.VectorSubcoreMesh(core_axis_name="c", subcore_axis_name="s")

    @functools.partial(
        pl.kernel,
        out_type=jax.ShapeDtypeStruct((N, C), jnp.float32),
        mesh=mesh,
        scratch_types=[
            pltpu.VMEM((TPW,), jnp.int32),
            pltpu.VMEM((TPW,), jnp.int32),
            pltpu.VMEM((GC, C), jnp.float32),
            pltpu.VMEM((GC, C), jnp.float32),
            pltpu.SemaphoreType.DMA,
            pltpu.SemaphoreType.DMA,
        ],
    )
    def combine(y_hbm, p1_hbm, p2_hbm, out_hbm, p1b, p2b, buf1, buf2,
                sem1, sem2):
        wid = lax.axis_index("s") * NC + lax.axis_index("c")
        t0 = wid * TPW
        pltpu.sync_copy(p1_hbm.at[pl.ds(t0, TPW)], p1b)
        pltpu.sync_copy(p2_hbm.at[pl.ds(t0, TPW)], p2b)
        for h in range(TPW // GC):
            c1 = pltpu.async_copy(y_hbm.at[p1b.at[pl.ds(h * GC, GC)]], buf1,
                                  sem1)
            c2 = pltpu.async_copy(y_hbm.at[p2b.at[pl.ds(h * GC, GC)]], buf2,
                                  sem2)
            c1.wait()
            c2.wait()

            def _add(j, carry):
                for k in range(CV):
                    buf1[j, pl.ds(k * L, L)] = (buf1[j, pl.ds(k * L, L)]
                                                + buf2[j, pl.ds(k * L, L)])
                return carry

            lax.fori_loop(0, GC, _add, 0)
            pltpu.sync_copy(buf1, out_hbm.at[pl.ds(t0 + h * GC, GC)])

    return combine


def kernel(x, noise, Wr, br, Wn, bn, W1, b1, W2, b2):
    Bb, Tt, C = x.shape
    N = Bb * Tt
    E = Wr.shape[1]
    F = W1.shape[2]
    CAP = (N * TOP_K) // E

    xf = x.reshape(N, C)
    noiseT = noise.reshape(N, E).T
    e1, e2, g1, g2 = _router_call(
        xf, Wr.T, br.reshape(E, 1), Wn.T, bn.reshape(E, 1), noiseT)

    compact = _make_compact_kernel(N, E, CAP)
    tok_map, gate_map, slot_mat, counts = compact(e1, e2, g1, g2)

    dispatch = _make_dispatch_kernel(N, C, E, CAP)
    xe, p1, p2 = dispatch(xf, e1, e2, tok_map, slot_mat, counts)

    y = _ffn_call(xe, W1, b1, W2, b2, gate_map, CAP, FB=768)

    combine = _make_combine_kernel(N, C, E * CAP)
    out = combine(y, p1, p2)
    return out.reshape(Bb, Tt, C)


# speedup vs baseline: 3.3218x; 1.0018x over previous
import functools

import jax
import jax.numpy as jnp
from jax import lax
from jax.experimental import pallas as pl
from jax.experimental.pallas import tpu as pltpu
from jax.experimental.pallas import tpu_sc as plsc

TOP_K = 2
NC, NS, L = 2, 16, 16
NW = NC * NS


def _router_body(x_ref, wrt_ref, br_ref, wnt_ref, bn_ref, noiset_ref,
                 e1_ref, e2_ref, g1_ref, g2_ref):
    x = x_ref[...]
    dn = (((1,), (1,)), ((), ()))
    lg = lax.dot_general(wrt_ref[...], x, dn,
                         preferred_element_type=jnp.float32) + br_ref[...]
    nl = lax.dot_general(wnt_ref[...], x, dn,
                         preferred_element_type=jnp.float32) + bn_ref[...]
    sp = jnp.maximum(nl, 0.0) + jnp.log(1.0 + jnp.exp(-jnp.abs(nl)))
    noisy = lg + noiset_ref[...] * sp

    E = noisy.shape[0]
    iota = lax.broadcasted_iota(jnp.int32, noisy.shape, 0)
    m1 = jnp.max(noisy, axis=0)
    e1 = jnp.min(jnp.where(noisy == m1[None, :], iota, E), axis=0)
    masked = jnp.where(iota == e1[None, :], -jnp.inf, noisy)
    m2 = jnp.max(masked, axis=0)
    e2 = jnp.min(jnp.where(masked == m2[None, :], iota, E), axis=0)
    z = jnp.exp(m2 - m1)
    denom = 1.0 + z
    e1_ref[...] = e1[None, :]
    e2_ref[...] = e2[None, :]
    g1_ref[...] = (1.0 / denom)[None, :]
    g2_ref[...] = (z / denom)[None, :]


def _router_call(xf, WrT, brc, WnT, bnc, noiseT):
    N = xf.shape[0]
    return pl.pallas_call(
        _router_body,
        out_shape=(
            jax.ShapeDtypeStruct((1, N), jnp.int32),
            jax.ShapeDtypeStruct((1, N), jnp.int32),
            jax.ShapeDtypeStruct((1, N), jnp.float32),
            jax.ShapeDtypeStruct((1, N), jnp.float32),
        ),
    )(xf, WrT, brc, WnT, bnc, noiseT)


def _make_compact_kernel(N, E, CAP):
    NCHUNK = N // L
    mesh = plsc.VectorSubcoreMesh(core_axis_name="c", subcore_axis_name="s")

    @functools.partial(
        pl.kernel,
        out_type=(
            jax.ShapeDtypeStruct((E, CAP), jnp.int32),
            jax.ShapeDtypeStruct((E, CAP), jnp.float32),
            jax.ShapeDtypeStruct((E, N), jnp.int32),
            jax.ShapeDtypeStruct((E, L), jnp.int32),
        ),
        mesh=mesh,
        compiler_params=pltpu.CompilerParams(needs_layout_passes=False),
        scratch_types=[
            pltpu.VMEM((N,), jnp.int32),
            pltpu.VMEM((N,), jnp.int32),
            pltpu.VMEM((N,), jnp.float32),
            pltpu.VMEM((N,), jnp.float32),
            pltpu.VMEM((N + L,), jnp.int32),
            pltpu.VMEM((N + L,), jnp.float32),
            pltpu.VMEM((N,), jnp.int32),
            pltpu.VMEM((L,), jnp.int32),
        ],
    )
    def compact(e1_hbm, e2_hbm, g1_hbm, g2_hbm,
                tok_hbm, gate_hbm, slot_hbm, cnt_hbm,
                e1b, e2b, g1b, g2b, tokb, gateb, slotb, cntb):
        wid = lax.axis_index("s") * NC + lax.axis_index("c")

        @pl.when(wid < E)
        def _():
            eid = wid
            pltpu.sync_copy(e1_hbm.at[0], e1b)
            pltpu.sync_copy(e2_hbm.at[0], e2b)
            pltpu.sync_copy(g1_hbm.at[0], g1b)
            pltpu.sync_copy(g2_hbm.at[0], g2b)

            zi = jnp.zeros((L,), jnp.int32)
            zf = jnp.zeros((L,), jnp.float32)

            def _zero(i, carry):
                tokb[pl.ds(i * L, L)] = zi
                gateb[pl.ds(i * L, L)] = zf
                return carry

            lax.fori_loop(0, CAP // L, _zero, 0)

            iota = lax.iota(jnp.int32, L)

            def _scan(c, off):
                ve1 = e1b[pl.ds(c * L, L)]
                ve2 = e2b[pl.ds(c * L, L)]
                m1 = ve1 == eid
                m2 = ve2 == eid
                mask = jnp.logical_or(m1, m2)
                mi = mask.astype(jnp.int32)
                inc = plsc.cumsum(mi)
                slotv = off + (inc - mi)
                slotb[pl.ds(c * L, L)] = slotv
                g = jnp.where(m1, g1b[pl.ds(c * L, L)],
                              jnp.where(m2, g2b[pl.ds(c * L, L)], 0.0))
                tokv = c * L + iota
                plsc.store_compressed(tokb.at[pl.ds(off, L)], tokv, mask=mask)
                plsc.store_compressed(gateb.at[pl.ds(off, L)], g, mask=mask)
                return off + jnp.sum(mi)

            cnt = lax.fori_loop(0, NCHUNK, _scan, jnp.int32(0))

            pltpu.sync_copy(tokb.at[pl.ds(0, CAP)], tok_hbm.at[eid])
            pltpu.sync_copy(gateb.at[pl.ds(0, CAP)], gate_hbm.at[eid])
            pltpu.sync_copy(slotb, slot_hbm.at[eid])
            cntb[pl.ds(0, L)] = jnp.full((L,), cnt, jnp.int32)
            pltpu.sync_copy(cntb, cnt_hbm.at[eid])

    return compact


def _make_dispatch_kernel(N, C, E, CAP):
    TPW = N // NW
    NCH = TPW // L
    RPW = (E * CAP) // NW
    GC = 64
    TPE = NW // E
    mesh = plsc.VectorSubcoreMesh(core_axis_name="c", subcore_axis_name="s")

    @functools.partial(
        pl.kernel,
        out_type=(
            jax.ShapeDtypeStruct((E * CAP, C), jnp.float32),
            jax.ShapeDtypeStruct((N,), jnp.int32),
            jax.ShapeDtypeStruct((N,), jnp.int32),
        ),
        mesh=mesh,
        scratch_types=[
            pltpu.VMEM((E, TPW), jnp.int32),
            pltpu.VMEM((E, L), jnp.int32),
            pltpu.VMEM((TPW,), jnp.int32),
            pltpu.VMEM((TPW,), jnp.int32),
            pltpu.VMEM((TPW,), jnp.int32),
            pltpu.VMEM((TPW,), jnp.int32),
            pltpu.VMEM((RPW,), jnp.int32),
            pltpu.VMEM((GC, C), jnp.float32),
            pltpu.SemaphoreType.DMA,
        ],
    )
    def dispatch(x_hbm, e1_hbm, e2_hbm, tok_hbm, slot_hbm, cnt_hbm,
                 xe_hbm, p1_hbm, p2_hbm,
                 slotm, cnts, e1b, e2b, p1b, p2b, idxb, rowb, sem):
        wid = lax.axis_index("s") * NC + lax.axis_index("c")
        t0 = wid * TPW

        pltpu.sync_copy(slot_hbm.at[:, pl.ds(t0, TPW)], slotm)
        pltpu.sync_copy(cnt_hbm, cnts)
        pltpu.sync_copy(e1_hbm.at[0, pl.ds(t0, TPW)], e1b)
        pltpu.sync_copy(e2_hbm.at[0, pl.ds(t0, TPW)], e2b)

        ez = jnp.int32(-1)
        for e in range(E):
            tot = cnts[e][0]
            take = jnp.logical_and(tot < CAP, ez < 0)
            ez = jnp.where(take, jnp.int32(e), ez)
        zero_flat = jnp.where(ez >= 0, ez * CAP + (CAP - 1), 0)

        for c in range(NCH):
            ve1 = e1b[pl.ds(c * L, L)]
            ve2 = e2b[pl.ds(c * L, L)]
            s1 = jnp.zeros((L,), jnp.int32)
            s2 = jnp.zeros((L,), jnp.int32)
            for e in range(E):
                row = slotm[e, pl.ds(c * L, L)]
                s1 = jnp.where(ve1 == e, row, s1)
                s2 = jnp.where(ve2 == e, row, s2)
            p1v = jnp.where(s1 < CAP, ve1 * CAP + s1, zero_flat)
            p2v = jnp.where(s2 < CAP, ve2 * CAP + s2, zero_flat)
            p1b[pl.ds(c * L, L)] = p1v
            p2b[pl.ds(c * L, L)] = p2v

        pltpu.sync_copy(p1b, p1_hbm.at[pl.ds(t0, TPW)])
        pltpu.sync_copy(p2b, p2_hbm.at[pl.ds(t0, TPW)])

        eid = wid // TPE
        s0 = (wid % TPE) * RPW
        pltpu.sync_copy(tok_hbm.at[eid, pl.ds(s0, RPW)], idxb)
        r0 = wid * RPW
        for k in range(RPW // GC):
            pltpu.async_copy(x_hbm.at[idxb.at[pl.ds(k * GC, GC)]], rowb,
                             sem).wait()
            pltpu.sync_copy(rowb, xe_hbm.at[pl.ds(r0 + k * GC, GC)])

    return dispatch


def _ffn_body(xe_ref, w1_ref, b1_ref, w2_ref, b2_ref, gate_ref, y_ref,
              acc_ref):
    f = pl.program_id(1)
    nf = pl.num_programs(1)
    h = jnp.dot(xe_ref[...].astype(jnp.bfloat16),
                w1_ref[0].astype(jnp.bfloat16),
                preferred_element_type=jnp.float32)
    h = jnp.maximum(h + b1_ref[0], 0.0)
    part = jnp.dot(h.astype(jnp.bfloat16), w2_ref[0].astype(jnp.bfloat16),
                   preferred_element_type=jnp.float32)

    @pl.when(f == 0)
    def _():
        acc_ref[...] = jnp.zeros_like(acc_ref)

    acc_ref[...] += part

    @pl.when(f == nf - 1)
    def _():
        y_ref[...] = (acc_ref[...] + b2_ref[0]) * gate_ref[...]


def _ffn_call(xe, W1, b1, W2, b2, gate_map, CAP, FB):
    E, C, F = W1.shape
    grid = (E, F // FB)
    return pl.pallas_call(
        _ffn_body,
        grid=grid,
        in_specs=[
            pl.BlockSpec((CAP, C), lambda e, f: (e, 0)),
            pl.BlockSpec((1, C, FB), lambda e, f: (e, 0, f)),
            pl.BlockSpec((1, 1, FB), lambda e, f: (e, 0, f)),
            pl.BlockSpec((1, FB, C), lambda e, f: (e, f, 0)),
            pl.BlockSpec((1, 1, C), lambda e, f: (e, 0, 0)),
            pl.BlockSpec((CAP, 1), lambda e, f: (e, 0)),
        ],
        out_specs=pl.BlockSpec((CAP, C), lambda e, f: (e, 0)),
        out_shape=jax.ShapeDtypeStruct((E * CAP, C), jnp.float32),
        scratch_shapes=[pltpu.VMEM((CAP, C), jnp.float32)],
        compiler_params=pltpu.CompilerParams(
            dimension_semantics=("arbitrary", "arbitrary")),
    )(xe, W1, b1.reshape(E, 1, F), W2, b2.reshape(E, 1, C),
      gate_map.reshape(E * CAP, 1))


def _make_combine_kernel(N, C, YROWS):
    TPW = N // NW
    GC = 64
    CV = C // L
    mesh = plsc.VectorSubcoreMesh(core_axis_name="c", subcore_axis_name="s")

    @functools.partial(
        pl.kernel,
        out_type=jax.ShapeDtypeStruct((N, C), jnp.float32),
        mesh=mesh,
        scratch_types=[
            pltpu.VMEM((TPW,), jnp.int32),
            pltpu.VMEM((TPW,), jnp.int32),
            pltpu.VMEM((GC, C), jnp.float32),
            pltpu.VMEM((GC, C), jnp.float32),
            pltpu.SemaphoreType.DMA,
            pltpu.SemaphoreType.DMA,
        ],
    )
    def combine(y_hbm, p1_hbm, p2_hbm, out_hbm, p1b, p2b, buf1, buf2,
                sem1, sem2):
        wid = lax.axis_index("s") * NC + lax.axis_index("c")
        t0 = wid * TPW
        pltpu.sync_copy(p1_hbm.at[pl.ds(t0, TPW)], p1b)
        pltpu.sync_copy(p2_hbm.at[pl.ds(t0, TPW)], p2b)
        for h in range(TPW // GC):
            c1 = pltpu.async_copy(y_hbm.at[p1b.at[pl.ds(h * GC, GC)]], buf1,
                                  sem1)
            c2 = pltpu.async_copy(y_hbm.at[p2b.at[pl.ds(h * GC, GC)]], buf2,
                                  sem2)
            c1.wait()
            c2.wait()

            def _add(j, carry):
                for k in range(CV):
                    buf1[j, pl.ds(k * L, L)] = (buf1[j, pl.ds(k * L, L)]
                                                + buf2[j, pl.ds(k * L, L)])
                return carry

            lax.fori_loop(0, GC, _add, 0)
            pltpu.sync_copy(buf1, out_hbm.at[pl.ds(t0 + h * GC, GC)])

    return combine


def kernel(x, noise, Wr, br, Wn, bn, W1, b1, W2, b2):
    Bb, Tt, C = x.shape
    N = Bb * Tt
    E = Wr.shape[1]
    F = W1.shape[2]
    CAP = (N * TOP_K) // E

    xf = x.reshape(N, C)
    noiseT = noise.reshape(N, E).T
    e1, e2, g1, g2 = _router_call(
        xf, Wr.T, br.reshape(E, 1), Wn.T, bn.reshape(E, 1), noiseT)

    compact = _make_compact_kernel(N, E, CAP)
    tok_map, gate_map, slot_mat, counts = compact(e1, e2, g1, g2)

    dispatch = _make_dispatch_kernel(N, C, E, CAP)
    xe, p1, p2 = dispatch(xf, e1, e2, tok_map, slot_mat, counts)

    y = _ffn_call(xe, W1, b1, W2, b2, gate_map, CAP, FB=768)

    combine = _make_combine_kernel(N, C, E * CAP)
    out = combine(y, p1, p2)
    return out.reshape(Bb, Tt, C)


# speedup vs baseline: 3.4099x; 1.0265x over previous
import functools

import jax
import jax.numpy as jnp
from jax import lax
from jax.experimental import pallas as pl
from jax.experimental.pallas import tpu as pltpu
from jax.experimental.pallas import tpu_sc as plsc

TOP_K = 2
NC, NS, L = 2, 16, 16
NW = NC * NS


def _router_body(x_ref, wrt_ref, br_ref, wnt_ref, bn_ref, noiset_ref,
                 e1_ref, e2_ref, g1_ref, g2_ref):
    x = x_ref[...]
    dn = (((1,), (1,)), ((), ()))
    lg = lax.dot_general(wrt_ref[...], x, dn,
                         preferred_element_type=jnp.float32) + br_ref[...]
    nl = lax.dot_general(wnt_ref[...], x, dn,
                         preferred_element_type=jnp.float32) + bn_ref[...]
    sp = jnp.maximum(nl, 0.0) + jnp.log(1.0 + jnp.exp(-jnp.abs(nl)))
    noisy = lg + noiset_ref[...] * sp

    E = noisy.shape[0]
    iota = lax.broadcasted_iota(jnp.int32, noisy.shape, 0)
    m1 = jnp.max(noisy, axis=0)
    e1 = jnp.min(jnp.where(noisy == m1[None, :], iota, E), axis=0)
    masked = jnp.where(iota == e1[None, :], -jnp.inf, noisy)
    m2 = jnp.max(masked, axis=0)
    e2 = jnp.min(jnp.where(masked == m2[None, :], iota, E), axis=0)
    z = jnp.exp(m2 - m1)
    denom = 1.0 + z
    e1_ref[...] = e1[None, :]
    e2_ref[...] = e2[None, :]
    g1_ref[...] = (1.0 / denom)[None, :]
    g2_ref[...] = (z / denom)[None, :]


def _router_call(xf, WrT, brc, WnT, bnc, noiseT):
    N = xf.shape[0]
    return pl.pallas_call(
        _router_body,
        out_shape=(
            jax.ShapeDtypeStruct((1, N), jnp.int32),
            jax.ShapeDtypeStruct((1, N), jnp.int32),
            jax.ShapeDtypeStruct((1, N), jnp.float32),
            jax.ShapeDtypeStruct((1, N), jnp.float32),
        ),
    )(xf, WrT, brc, WnT, bnc, noiseT)


def _make_dispatch_kernel(N, C, E, CAP):
    NCHUNK = N // L
    EPC = E // NC
    TPE = NS // EPC
    RPT = CAP // TPE
    GC = 32
    NGC = RPT // GC
    mesh = plsc.VectorSubcoreMesh(core_axis_name="c", subcore_axis_name="s")

    @functools.partial(
        pl.kernel,
        out_type=(
            jax.ShapeDtypeStruct((E * CAP, C), jnp.float32),
            jax.ShapeDtypeStruct((E, CAP), jnp.float32),
            jax.ShapeDtypeStruct((E, N), jnp.int32),
            jax.ShapeDtypeStruct((E, L), jnp.int32),
        ),
        mesh=mesh,
        compiler_params=pltpu.CompilerParams(needs_layout_passes=False),
        scratch_types=[
            pltpu.VMEM((N,), jnp.int32),
            pltpu.VMEM((N,), jnp.int32),
            pltpu.VMEM((N,), jnp.float32),
            pltpu.VMEM((N,), jnp.float32),
            pltpu.VMEM((N + L,), jnp.int32),
            pltpu.VMEM((N + L,), jnp.float32),
            pltpu.VMEM((N,), jnp.int32),
            pltpu.VMEM((L,), jnp.int32),
            pltpu.VMEM((RPT,), jnp.int32),
            pltpu.VMEM((GC, C), jnp.float32),
            pltpu.VMEM((GC, C), jnp.float32),
            pltpu.VMEM_SHARED((EPC, CAP), jnp.int32),
            pltpu.SemaphoreType.DMA,
            pltpu.SemaphoreType.DMA,
            pltpu.SemaphoreType.DMA,
            pltpu.SemaphoreType.DMA,
        ],
    )
    def dispatch(x_hbm, e1_hbm, e2_hbm, g1_hbm, g2_hbm,
                 xe_hbm, gate_hbm, slot_hbm, cnt_hbm,
                 e1b, e2b, g1b, g2b, tokb, gateb, slotb, cntb,
                 idxb, rowa, rowb, sh_tok,
                 gsem0, gsem1, wsem0, wsem1):
        cid = lax.axis_index("c")
        sid = lax.axis_index("s")

        @pl.when(sid < EPC)
        def _():
            eid = cid * EPC + sid
            pltpu.sync_copy(e1_hbm.at[0], e1b)
            pltpu.sync_copy(e2_hbm.at[0], e2b)
            pltpu.sync_copy(g1_hbm.at[0], g1b)
            pltpu.sync_copy(g2_hbm.at[0], g2b)

            zi = jnp.zeros((L,), jnp.int32)
            zf = jnp.zeros((L,), jnp.float32)

            def _zero(i, carry):
                tokb[pl.ds(i * L, L)] = zi
                gateb[pl.ds(i * L, L)] = zf
                return carry

            lax.fori_loop(0, CAP // L, _zero, 0)

            iota = lax.iota(jnp.int32, L)

            def _scan(c, off):
                ve1 = e1b[pl.ds(c * L, L)]
                ve2 = e2b[pl.ds(c * L, L)]
                m1 = ve1 == eid
                m2 = ve2 == eid
                mask = jnp.logical_or(m1, m2)
                mi = mask.astype(jnp.int32)
                inc = plsc.cumsum(mi)
                slotv = off + (inc - mi)
                slotb[pl.ds(c * L, L)] = slotv
                g = jnp.where(m1, g1b[pl.ds(c * L, L)],
                              jnp.where(m2, g2b[pl.ds(c * L, L)], 0.0))
                tokv = c * L + iota
                plsc.store_compressed(tokb.at[pl.ds(off, L)], tokv, mask=mask)
                plsc.store_compressed(gateb.at[pl.ds(off, L)], g, mask=mask)
                return off + jnp.sum(mi)

            cnt = lax.fori_loop(0, NCHUNK, _scan, jnp.int32(0))

            pltpu.sync_copy(tokb.at[pl.ds(0, CAP)], sh_tok.at[sid])
            pltpu.sync_copy(gateb.at[pl.ds(0, CAP)], gate_hbm.at[eid])
            pltpu.sync_copy(slotb, slot_hbm.at[eid])
            cntb[pl.ds(0, L)] = jnp.full((L,), cnt, jnp.int32)
            pltpu.sync_copy(cntb, cnt_hbm.at[eid])

        plsc.subcore_barrier()

        el = sid // TPE
        s0 = (sid % TPE) * RPT
        pltpu.sync_copy(sh_tok.at[el, pl.ds(s0, RPT)], idxb)
        r0 = (cid * EPC + el) * CAP + s0
        bufs = (rowa, rowb)
        gsems = (gsem0, gsem1)
        wsems = (wsem0, wsem1)
        gd = [None] * NGC
        wd = [None] * NGC
        gd[0] = pltpu.async_copy(x_hbm.at[idxb.at[pl.ds(0, GC)]], bufs[0],
                                 gsems[0])
        for k in range(NGC):
            b = k % 2
            if k + 1 < NGC:
                if k - 1 >= 0:
                    wd[k - 1].wait()
                gd[k + 1] = pltpu.async_copy(
                    x_hbm.at[idxb.at[pl.ds((k + 1) * GC, GC)]],
                    bufs[(k + 1) % 2], gsems[(k + 1) % 2])
            gd[k].wait()
            wd[k] = pltpu.async_copy(bufs[b],
                                     xe_hbm.at[pl.ds(r0 + k * GC, GC)],
                                     wsems[b])
        wd[NGC - 2].wait()
        wd[NGC - 1].wait()

    return dispatch


def _ffn_body(xe_ref, w1_ref, b1_ref, w2_ref, b2_ref, gate_ref, y_ref,
              acc_ref):
    f = pl.program_id(1)
    nf = pl.num_programs(1)
    h = jnp.dot(xe_ref[...].astype(jnp.bfloat16),
                w1_ref[0].astype(jnp.bfloat16),
                preferred_element_type=jnp.float32)
    h = jnp.maximum(h + b1_ref[0], 0.0)
    part = jnp.dot(h.astype(jnp.bfloat16), w2_ref[0].astype(jnp.bfloat16),
                   preferred_element_type=jnp.float32)

    @pl.when(f == 0)
    def _():
        acc_ref[...] = jnp.zeros_like(acc_ref)

    acc_ref[...] += part

    @pl.when(f == nf - 1)
    def _():
        y_ref[...] = (acc_ref[...] + b2_ref[0]) * gate_ref[...]


def _ffn_call(xe, W1, b1, W2, b2, gate_map, CAP, FB):
    E, C, F = W1.shape
    grid = (E, F // FB)
    return pl.pallas_call(
        _ffn_body,
        grid=grid,
        in_specs=[
            pl.BlockSpec((CAP, C), lambda e, f: (e, 0)),
            pl.BlockSpec((1, C, FB), lambda e, f: (e, 0, f)),
            pl.BlockSpec((1, 1, FB), lambda e, f: (e, 0, f)),
            pl.BlockSpec((1, FB, C), lambda e, f: (e, f, 0)),
            pl.BlockSpec((1, 1, C), lambda e, f: (e, 0, 0)),
            pl.BlockSpec((CAP, 1), lambda e, f: (e, 0)),
        ],
        out_specs=pl.BlockSpec((CAP, C), lambda e, f: (e, 0)),
        out_shape=jax.ShapeDtypeStruct((E * CAP, C), jnp.float32),
        scratch_shapes=[pltpu.VMEM((CAP, C), jnp.float32)],
        compiler_params=pltpu.CompilerParams(
            dimension_semantics=("arbitrary", "arbitrary")),
    )(xe, W1, b1.reshape(E, 1, F), W2, b2.reshape(E, 1, C),
      gate_map.reshape(E * CAP, 1))


def _make_combine_kernel(N, C, E, CAP):
    TPW = N // NW
    NCH = TPW // L
    GC = 32
    NGC = TPW // GC
    CV = C // L
    mesh = plsc.VectorSubcoreMesh(core_axis_name="c", subcore_axis_name="s")

    @functools.partial(
        pl.kernel,
        out_type=jax.ShapeDtypeStruct((N, C), jnp.float32),
        mesh=mesh,
        compiler_params=pltpu.CompilerParams(needs_layout_passes=False),
        scratch_types=[
            pltpu.VMEM((E, TPW), jnp.int32),
            pltpu.VMEM((E, L), jnp.int32),
            pltpu.VMEM((TPW,), jnp.int32),
            pltpu.VMEM((TPW,), jnp.int32),
            pltpu.VMEM((TPW,), jnp.int32),
            pltpu.VMEM((TPW,), jnp.int32),
            pltpu.VMEM((GC, C), jnp.float32),
            pltpu.VMEM((GC, C), jnp.float32),
            pltpu.VMEM((GC, C), jnp.float32),
            pltpu.VMEM((GC, C), jnp.float32),
            pltpu.SemaphoreType.DMA,
            pltpu.SemaphoreType.DMA,
            pltpu.SemaphoreType.DMA,
            pltpu.SemaphoreType.DMA,
            pltpu.SemaphoreType.DMA,
            pltpu.SemaphoreType.DMA,
        ],
    )
    def combine(y_hbm, slot_hbm, cnt_hbm, e1_hbm, e2_hbm, out_hbm,
                slotm, cnts, e1b, e2b, p1b, p2b,
                a1, a2, b1, b2, ga1, ga2, gb1, gb2, wsa, wsb):
        cid = lax.axis_index("c")
        sid = lax.axis_index("s")
        wid = sid * NC + cid
        t0 = wid * TPW

        pltpu.sync_copy(slot_hbm.at[:, pl.ds(t0, TPW)], slotm)
        pltpu.sync_copy(cnt_hbm, cnts)
        pltpu.sync_copy(e1_hbm.at[0, pl.ds(t0, TPW)], e1b)
        pltpu.sync_copy(e2_hbm.at[0, pl.ds(t0, TPW)], e2b)

        ez = jnp.int32(-1)
        for e in range(E):
            tot = cnts[e][0]
            take = jnp.logical_and(tot < CAP, ez < 0)
            ez = jnp.where(take, jnp.int32(e), ez)
        zero_flat = jnp.where(ez >= 0, ez * CAP + (CAP - 1), 0)

        for c in range(NCH):
            ve1 = e1b[pl.ds(c * L, L)]
            ve2 = e2b[pl.ds(c * L, L)]
            s1 = jnp.zeros((L,), jnp.int32)
            s2 = jnp.zeros((L,), jnp.int32)
            for e in range(E):
                row = slotm[e, pl.ds(c * L, L)]
                s1 = jnp.where(ve1 == e, row, s1)
                s2 = jnp.where(ve2 == e, row, s2)
            p1b[pl.ds(c * L, L)] = jnp.where(s1 < CAP, ve1 * CAP + s1,
                                             zero_flat)
            p2b[pl.ds(c * L, L)] = jnp.where(s2 < CAP, ve2 * CAP + s2,
                                             zero_flat)

        sets = ((a1, a2, ga1, ga2, wsa), (b1, b2, gb1, gb2, wsb))

        def fire(k):
            u1, u2, s1_, s2_, _ = sets[k % 2]
            d1 = pltpu.async_copy(y_hbm.at[p1b.at[pl.ds(k * GC, GC)]], u1,
                                  s1_)
            d2 = pltpu.async_copy(y_hbm.at[p2b.at[pl.ds(k * GC, GC)]], u2,
                                  s2_)
            return (d1, d2)

        gd = [None] * NGC
        wd = [None] * NGC
        gd[0] = fire(0)
        for k in range(NGC):
            u1, u2, _, _, ws = sets[k % 2]
            if k + 1 < NGC:
                if k - 1 >= 0:
                    wd[k - 1].wait()
                gd[k + 1] = fire(k + 1)
            gd[k][0].wait()
            gd[k][1].wait()

            def _add(j, carry):
                for v in range(CV):
                    u1[j, pl.ds(v * L, L)] = (u1[j, pl.ds(v * L, L)]
                                              + u2[j, pl.ds(v * L, L)])
                return carry

            lax.fori_loop(0, GC, _add, 0)
            wd[k] = pltpu.async_copy(u1, out_hbm.at[pl.ds(t0 + k * GC, GC)],
                                     ws)
        wd[NGC - 2].wait()
        wd[NGC - 1].wait()

    return combine


def kernel(x, noise, Wr, br, Wn, bn, W1, b1, W2, b2):
    Bb, Tt, C = x.shape
    N = Bb * Tt
    E = Wr.shape[1]
    CAP = (N * TOP_K) // E

    xf = x.reshape(N, C)
    noiseT = noise.reshape(N, E).T
    e1, e2, g1, g2 = _router_call(
        xf, Wr.T, br.reshape(E, 1), Wn.T, bn.reshape(E, 1), noiseT)

    dispatch = _make_dispatch_kernel(N, C, E, CAP)
    xe, gate_map, slot_mat, counts = dispatch(xf, e1, e2, g1, g2)

    y = _ffn_call(xe, W1, b1, W2, b2, gate_map, CAP, FB=768)

    combine = _make_combine_kernel(N, C, E, CAP)
    out = combine(y, slot_mat, counts, e1, e2)
    return out.reshape(Bb, Tt, C)


# speedup vs baseline: 3.8007x; 1.1146x over previous
import functools

import jax
import jax.numpy as jnp
from jax import lax
from jax.experimental import pallas as pl
from jax.experimental.pallas import tpu as pltpu
from jax.experimental.pallas import tpu_sc as plsc

TOP_K = 2
NC, NS, L = 2, 16, 16
NW = NC * NS


def _router_body(x_ref, wrt_ref, br_ref, wnt_ref, bn_ref, noiset_ref,
                 e1_ref, e2_ref, g1_ref, g2_ref):
    x = x_ref[...]
    dn = (((1,), (1,)), ((), ()))
    lg = lax.dot_general(wrt_ref[...], x, dn,
                         preferred_element_type=jnp.float32) + br_ref[...]
    nl = lax.dot_general(wnt_ref[...], x, dn,
                         preferred_element_type=jnp.float32) + bn_ref[...]
    sp = jnp.maximum(nl, 0.0) + jnp.log(1.0 + jnp.exp(-jnp.abs(nl)))
    noisy = lg + noiset_ref[...] * sp

    E = noisy.shape[0]
    iota = lax.broadcasted_iota(jnp.int32, noisy.shape, 0)
    m1 = jnp.max(noisy, axis=0)
    e1 = jnp.min(jnp.where(noisy == m1[None, :], iota, E), axis=0)
    masked = jnp.where(iota == e1[None, :], -jnp.inf, noisy)
    m2 = jnp.max(masked, axis=0)
    e2 = jnp.min(jnp.where(masked == m2[None, :], iota, E), axis=0)
    z = jnp.exp(m2 - m1)
    denom = 1.0 + z
    e1_ref[...] = e1[None, :]
    e2_ref[...] = e2[None, :]
    g1_ref[...] = (1.0 / denom)[None, :]
    g2_ref[...] = (z / denom)[None, :]


def _router_call(xf, WrT, brc, WnT, bnc, noiseT):
    N = xf.shape[0]
    return pl.pallas_call(
        _router_body,
        out_shape=(
            jax.ShapeDtypeStruct((1, N), jnp.int32),
            jax.ShapeDtypeStruct((1, N), jnp.int32),
            jax.ShapeDtypeStruct((1, N), jnp.float32),
            jax.ShapeDtypeStruct((1, N), jnp.float32),
        ),
    )(xf, WrT, brc, WnT, bnc, noiseT)


def _make_dispatch_kernel(N, C, E, CAP):
    NCHUNK = N // L
    EPC = E // NC
    TPE = NS // EPC
    RPT = CAP // TPE
    GC = 32
    NGC = RPT // GC
    mesh = plsc.VectorSubcoreMesh(core_axis_name="c", subcore_axis_name="s")

    @functools.partial(
        pl.kernel,
        out_type=(
            jax.ShapeDtypeStruct((E * CAP, C), jnp.float32),
            jax.ShapeDtypeStruct((E, CAP), jnp.float32),
            jax.ShapeDtypeStruct((E, N), jnp.int32),
            jax.ShapeDtypeStruct((E, L), jnp.int32),
        ),
        mesh=mesh,
        compiler_params=pltpu.CompilerParams(needs_layout_passes=False),
        scratch_types=[
            pltpu.VMEM((N,), jnp.int32),
            pltpu.VMEM((N,), jnp.int32),
            pltpu.VMEM((N,), jnp.float32),
            pltpu.VMEM((N,), jnp.float32),
            pltpu.VMEM((N + L,), jnp.int32),
            pltpu.VMEM((N + L,), jnp.float32),
            pltpu.VMEM((N,), jnp.int32),
            pltpu.VMEM((L,), jnp.int32),
            pltpu.VMEM((RPT,), jnp.int32),
            pltpu.VMEM((GC, C), jnp.float32),
            pltpu.VMEM((GC, C), jnp.float32),
            pltpu.VMEM_SHARED((EPC, CAP), jnp.int32),
            pltpu.SemaphoreType.DMA,
            pltpu.SemaphoreType.DMA,
            pltpu.SemaphoreType.DMA,
            pltpu.SemaphoreType.DMA,
        ],
    )
    def dispatch(x_hbm, e1_hbm, e2_hbm, g1_hbm, g2_hbm,
                 xe_hbm, gate_hbm, slot_hbm, cnt_hbm,
                 e1b, e2b, g1b, g2b, tokb, gateb, slotb, cntb,
                 idxb, rowa, rowb, sh_tok,
                 gsem0, gsem1, wsem0, wsem1):
        cid = lax.axis_index("c")
        sid = lax.axis_index("s")

        @pl.when(sid < EPC)
        def _():
            eid = cid * EPC + sid
            pltpu.sync_copy(e1_hbm.at[0], e1b)
            pltpu.sync_copy(e2_hbm.at[0], e2b)
            pltpu.sync_copy(g1_hbm.at[0], g1b)
            pltpu.sync_copy(g2_hbm.at[0], g2b)

            zi = jnp.zeros((L,), jnp.int32)
            zf = jnp.zeros((L,), jnp.float32)

            def _zero(i, carry):
                tokb[pl.ds(i * L, L)] = zi
                gateb[pl.ds(i * L, L)] = zf
                return carry

            lax.fori_loop(0, CAP // L, _zero, 0)

            iota = lax.iota(jnp.int32, L)

            def _scan(c, off):
                ve1 = e1b[pl.ds(c * L, L)]
                ve2 = e2b[pl.ds(c * L, L)]
                m1 = ve1 == eid
                m2 = ve2 == eid
                mask = jnp.logical_or(m1, m2)
                mi = mask.astype(jnp.int32)
                inc = plsc.cumsum(mi)
                slotv = off + (inc - mi)
                slotb[pl.ds(c * L, L)] = slotv
                g = jnp.where(m1, g1b[pl.ds(c * L, L)],
                              jnp.where(m2, g2b[pl.ds(c * L, L)], 0.0))
                tokv = c * L + iota
                plsc.store_compressed(tokb.at[pl.ds(off, L)], tokv, mask=mask)
                plsc.store_compressed(gateb.at[pl.ds(off, L)], g, mask=mask)
                return off + jnp.sum(mi)

            cnt = lax.fori_loop(0, NCHUNK, _scan, jnp.int32(0))

            pltpu.sync_copy(tokb.at[pl.ds(0, CAP)], sh_tok.at[sid])
            pltpu.sync_copy(gateb.at[pl.ds(0, CAP)], gate_hbm.at[eid])
            pltpu.sync_copy(slotb, slot_hbm.at[eid])
            cntb[pl.ds(0, L)] = jnp.full((L,), cnt, jnp.int32)
            pltpu.sync_copy(cntb, cnt_hbm.at[eid])

        plsc.subcore_barrier()

        el = sid // TPE
        s0 = (sid % TPE) * RPT
        pltpu.sync_copy(sh_tok.at[el, pl.ds(s0, RPT)], idxb)
        r0 = (cid * EPC + el) * CAP + s0
        bufs = (rowa, rowb)
        gsems = (gsem0, gsem1)
        wsems = (wsem0, wsem1)
        gd = [None] * NGC
        wd = [None] * NGC
        gd[0] = pltpu.async_copy(x_hbm.at[idxb.at[pl.ds(0, GC)]], bufs[0],
                                 gsems[0])
        for k in range(NGC):
            b = k % 2
            if k + 1 < NGC:
                if k - 1 >= 0:
                    wd[k - 1].wait()
                gd[k + 1] = pltpu.async_copy(
                    x_hbm.at[idxb.at[pl.ds((k + 1) * GC, GC)]],
                    bufs[(k + 1) % 2], gsems[(k + 1) % 2])
            gd[k].wait()
            wd[k] = pltpu.async_copy(bufs[b],
                                     xe_hbm.at[pl.ds(r0 + k * GC, GC)],
                                     wsems[b])
        wd[NGC - 2].wait()
        wd[NGC - 1].wait()

    return dispatch


def _ffn_body(xe_ref, w1_ref, b1_ref, w2_ref, b2_ref, gate_ref, y_ref):
    h = jnp.dot(xe_ref[...].astype(jnp.bfloat16),
                w1_ref[0].astype(jnp.bfloat16),
                preferred_element_type=jnp.float32)
    h = jnp.maximum(h + b1_ref[0], 0.0)
    part = jnp.dot(h.astype(jnp.bfloat16), w2_ref[0].astype(jnp.bfloat16),
                   preferred_element_type=jnp.float32)
    y_ref[...] = (part + b2_ref[0]) * gate_ref[...]


def _ffn_call(xe, W1, b1, W2, b2, gate_map, CAP):
    E, C, F = W1.shape
    return pl.pallas_call(
        _ffn_body,
        grid=(E,),
        in_specs=[
            pl.BlockSpec((CAP, C), lambda e: (e, 0)),
            pl.BlockSpec((1, C, F), lambda e: (e, 0, 0)),
            pl.BlockSpec((1, 1, F), lambda e: (e, 0, 0)),
            pl.BlockSpec((1, F, C), lambda e: (e, 0, 0)),
            pl.BlockSpec((1, 1, C), lambda e: (e, 0, 0)),
            pl.BlockSpec((CAP, 1), lambda e: (e, 0)),
        ],
        out_specs=pl.BlockSpec((CAP, C), lambda e: (e, 0)),
        out_shape=jax.ShapeDtypeStruct((E * CAP, C), jnp.float32),
        compiler_params=pltpu.CompilerParams(
            dimension_semantics=("arbitrary",),
            vmem_limit_bytes=110 * 1024 * 1024),
    )(xe, W1, b1.reshape(E, 1, F), W2, b2.reshape(E, 1, C),
      gate_map.reshape(E * CAP, 1))


def _make_combine_kernel(N, C, E, CAP):
    TPW = N // NW
    NCH = TPW // L
    GC = 32
    NGC = TPW // GC
    CV = C // L
    mesh = plsc.VectorSubcoreMesh(core_axis_name="c", subcore_axis_name="s")

    @functools.partial(
        pl.kernel,
        out_type=jax.ShapeDtypeStruct((N, C), jnp.float32),
        mesh=mesh,
        compiler_params=pltpu.CompilerParams(needs_layout_passes=False),
        scratch_types=[
            pltpu.VMEM((E, TPW), jnp.int32),
            pltpu.VMEM((E, L), jnp.int32),
            pltpu.VMEM((TPW,), jnp.int32),
            pltpu.VMEM((TPW,), jnp.int32),
            pltpu.VMEM((TPW,), jnp.int32),
            pltpu.VMEM((TPW,), jnp.int32),
            pltpu.VMEM((GC, C), jnp.float32),
            pltpu.VMEM((GC, C), jnp.float32),
            pltpu.VMEM((GC, C), jnp.float32),
            pltpu.VMEM((GC, C), jnp.float32),
            pltpu.SemaphoreType.DMA,
            pltpu.SemaphoreType.DMA,
            pltpu.SemaphoreType.DMA,
            pltpu.SemaphoreType.DMA,
            pltpu.SemaphoreType.DMA,
            pltpu.SemaphoreType.DMA,
        ],
    )
    def combine(y_hbm, slot_hbm, cnt_hbm, e1_hbm, e2_hbm, out_hbm,
                slotm, cnts, e1b, e2b, p1b, p2b,
                a1, a2, b1, b2, ga1, ga2, gb1, gb2, wsa, wsb):
        cid = lax.axis_index("c")
        sid = lax.axis_index("s")
        wid = sid * NC + cid
        t0 = wid * TPW

        pltpu.sync_copy(slot_hbm.at[:, pl.ds(t0, TPW)], slotm)
        pltpu.sync_copy(cnt_hbm, cnts)
        pltpu.sync_copy(e1_hbm.at[0, pl.ds(t0, TPW)], e1b)
        pltpu.sync_copy(e2_hbm.at[0, pl.ds(t0, TPW)], e2b)

        ez = jnp.int32(-1)
        for e in range(E):
            tot = cnts[e][0]
            take = jnp.logical_and(tot < CAP, ez < 0)
            ez = jnp.where(take, jnp.int32(e), ez)
        zero_flat = jnp.where(ez >= 0, ez * CAP + (CAP - 1), 0)

        for c in range(NCH):
            ve1 = e1b[pl.ds(c * L, L)]
            ve2 = e2b[pl.ds(c * L, L)]
            s1 = jnp.zeros((L,), jnp.int32)
            s2 = jnp.zeros((L,), jnp.int32)
            for e in range(E):
                row = slotm[e, pl.ds(c * L, L)]
                s1 = jnp.where(ve1 == e, row, s1)
                s2 = jnp.where(ve2 == e, row, s2)
            p1b[pl.ds(c * L, L)] = jnp.where(s1 < CAP, ve1 * CAP + s1,
                                             zero_flat)
            p2b[pl.ds(c * L, L)] = jnp.where(s2 < CAP, ve2 * CAP + s2,
                                             zero_flat)

        sets = ((a1, a2, ga1, ga2, wsa), (b1, b2, gb1, gb2, wsb))

        def fire(k):
            u1, u2, s1_, s2_, _ = sets[k % 2]
            d1 = pltpu.async_copy(y_hbm.at[p1b.at[pl.ds(k * GC, GC)]], u1,
                                  s1_)
            d2 = pltpu.async_copy(y_hbm.at[p2b.at[pl.ds(k * GC, GC)]], u2,
                                  s2_)
            return (d1, d2)

        gd = [None] * NGC
        wd = [None] * NGC
        gd[0] = fire(0)
        for k in range(NGC):
            u1, u2, _, _, ws = sets[k % 2]
            if k + 1 < NGC:
                if k - 1 >= 0:
                    wd[k - 1].wait()
                gd[k + 1] = fire(k + 1)
            gd[k][0].wait()
            gd[k][1].wait()

            def _add(j, carry):
                for v in range(CV):
                    u1[j, pl.ds(v * L, L)] = (u1[j, pl.ds(v * L, L)]
                                              + u2[j, pl.ds(v * L, L)])
                return carry

            lax.fori_loop(0, GC, _add, 0)
            wd[k] = pltpu.async_copy(u1, out_hbm.at[pl.ds(t0 + k * GC, GC)],
                                     ws)
        wd[NGC - 2].wait()
        wd[NGC - 1].wait()

    return combine


def kernel(x, noise, Wr, br, Wn, bn, W1, b1, W2, b2):
    Bb, Tt, C = x.shape
    N = Bb * Tt
    E = Wr.shape[1]
    CAP = (N * TOP_K) // E

    xf = x.reshape(N, C)
    noiseT = noise.reshape(N, E).T
    e1, e2, g1, g2 = _router_call(
        xf, Wr.T, br.reshape(E, 1), Wn.T, bn.reshape(E, 1), noiseT)

    dispatch = _make_dispatch_kernel(N, C, E, CAP)
    xe, gate_map, slot_mat, counts = dispatch(xf, e1, e2, g1, g2)

    y = _ffn_call(xe, W1, b1, W2, b2, gate_map, CAP)

    combine = _make_combine_kernel(N, C, E, CAP)
    out = combine(y, slot_mat, counts, e1, e2)
    return out.reshape(Bb, Tt, C)


# speedup vs baseline: 3.9483x; 1.0388x over previous
import functools

import jax
import jax.numpy as jnp
from jax import lax
from jax.experimental import pallas as pl
from jax.experimental.pallas import tpu as pltpu
from jax.experimental.pallas import tpu_sc as plsc

TOP_K = 2
NC, NS, L = 2, 16, 16
NW = NC * NS


def _router_body(x_ref, wrt_ref, br_ref, wnt_ref, bn_ref, noiset_ref,
                 e1_ref, e2_ref, g1_ref, g2_ref, xbf_ref):
    x = x_ref[...]
    dn = (((1,), (1,)), ((), ()))
    lg = lax.dot_general(wrt_ref[...], x, dn,
                         preferred_element_type=jnp.float32) + br_ref[...]
    nl = lax.dot_general(wnt_ref[...], x, dn,
                         preferred_element_type=jnp.float32) + bn_ref[...]
    sp = jnp.maximum(nl, 0.0) + jnp.log(1.0 + jnp.exp(-jnp.abs(nl)))
    noisy = lg + noiset_ref[...] * sp

    E = noisy.shape[0]
    iota = lax.broadcasted_iota(jnp.int32, noisy.shape, 0)
    m1 = jnp.max(noisy, axis=0)
    e1 = jnp.min(jnp.where(noisy == m1[None, :], iota, E), axis=0)
    masked = jnp.where(iota == e1[None, :], -jnp.inf, noisy)
    m2 = jnp.max(masked, axis=0)
    e2 = jnp.min(jnp.where(masked == m2[None, :], iota, E), axis=0)
    z = jnp.exp(m2 - m1)
    denom = 1.0 + z
    e1_ref[...] = e1[None, :]
    e2_ref[...] = e2[None, :]
    g1_ref[...] = (1.0 / denom)[None, :]
    g2_ref[...] = (z / denom)[None, :]
    CW = x.shape[1] // 2
    xb = x.astype(jnp.bfloat16)
    lo = lax.bitcast_convert_type(xb[:, :CW], jnp.uint16).astype(jnp.uint32)
    hi = lax.bitcast_convert_type(xb[:, CW:], jnp.uint16).astype(jnp.uint32)
    xbf_ref[...] = lax.bitcast_convert_type(lo | (hi << 16), jnp.int32)


def _router_call(xf, WrT, brc, WnT, bnc, noiseT):
    N = xf.shape[0]
    return pl.pallas_call(
        _router_body,
        out_shape=(
            jax.ShapeDtypeStruct((1, N), jnp.int32),
            jax.ShapeDtypeStruct((1, N), jnp.int32),
            jax.ShapeDtypeStruct((1, N), jnp.float32),
            jax.ShapeDtypeStruct((1, N), jnp.float32),
            jax.ShapeDtypeStruct((N, xf.shape[1] // 2), jnp.int32),
        ),
    )(xf, WrT, brc, WnT, bnc, noiseT)


def _make_dispatch_kernel(N, CW, E, CAP):
    NCHUNK = N // L
    EPC = E // NC
    TPE = NS // EPC
    RPT = CAP // TPE
    GC = 32
    NGC = RPT // GC
    mesh = plsc.VectorSubcoreMesh(core_axis_name="c", subcore_axis_name="s")

    @functools.partial(
        pl.kernel,
        out_type=(
            jax.ShapeDtypeStruct((E * CAP, CW), jnp.int32),
            jax.ShapeDtypeStruct((E, CAP), jnp.float32),
            jax.ShapeDtypeStruct((E, N), jnp.int32),
            jax.ShapeDtypeStruct((E, L), jnp.int32),
        ),
        mesh=mesh,
        compiler_params=pltpu.CompilerParams(needs_layout_passes=False),
        scratch_types=[
            pltpu.VMEM((N,), jnp.int32),
            pltpu.VMEM((N,), jnp.int32),
            pltpu.VMEM((N,), jnp.float32),
            pltpu.VMEM((N,), jnp.float32),
            pltpu.VMEM((N + L,), jnp.int32),
            pltpu.VMEM((N + L,), jnp.float32),
            pltpu.VMEM((N,), jnp.int32),
            pltpu.VMEM((L,), jnp.int32),
            pltpu.VMEM((RPT,), jnp.int32),
            pltpu.VMEM((GC, CW), jnp.int32),
            pltpu.VMEM((GC, CW), jnp.int32),
            pltpu.VMEM_SHARED((EPC, CAP), jnp.int32),
            pltpu.SemaphoreType.DMA,
            pltpu.SemaphoreType.DMA,
            pltpu.SemaphoreType.DMA,
            pltpu.SemaphoreType.DMA,
        ],
    )
    def dispatch(x_hbm, e1_hbm, e2_hbm, g1_hbm, g2_hbm,
                 xe_hbm, gate_hbm, slot_hbm, cnt_hbm,
                 e1b, e2b, g1b, g2b, tokb, gateb, slotb, cntb,
                 idxb, rowa, rowb, sh_tok,
                 gsem0, gsem1, wsem0, wsem1):
        cid = lax.axis_index("c")
        sid = lax.axis_index("s")

        @pl.when(sid < EPC)
        def _():
            eid = cid * EPC + sid
            pltpu.sync_copy(e1_hbm.at[0], e1b)
            pltpu.sync_copy(e2_hbm.at[0], e2b)
            pltpu.sync_copy(g1_hbm.at[0], g1b)
            pltpu.sync_copy(g2_hbm.at[0], g2b)

            zi = jnp.zeros((L,), jnp.int32)
            zf = jnp.zeros((L,), jnp.float32)

            def _zero(i, carry):
                tokb[pl.ds(i * L, L)] = zi
                gateb[pl.ds(i * L, L)] = zf
                return carry

            lax.fori_loop(0, CAP // L, _zero, 0)

            iota = lax.iota(jnp.int32, L)

            def _scan(c, off):
                ve1 = e1b[pl.ds(c * L, L)]
                ve2 = e2b[pl.ds(c * L, L)]
                m1 = ve1 == eid
                m2 = ve2 == eid
                mask = jnp.logical_or(m1, m2)
                mi = mask.astype(jnp.int32)
                inc = plsc.cumsum(mi)
                slotv = off + (inc - mi)
                slotb[pl.ds(c * L, L)] = slotv
                g = jnp.where(m1, g1b[pl.ds(c * L, L)],
                              jnp.where(m2, g2b[pl.ds(c * L, L)], 0.0))
                tokv = c * L + iota
                plsc.store_compressed(tokb.at[pl.ds(off, L)], tokv, mask=mask)
                plsc.store_compressed(gateb.at[pl.ds(off, L)], g, mask=mask)
                return off + jnp.sum(mi)

            cnt = lax.fori_loop(0, NCHUNK, _scan, jnp.int32(0))

            pltpu.sync_copy(tokb.at[pl.ds(0, CAP)], sh_tok.at[sid])
            pltpu.sync_copy(gateb.at[pl.ds(0, CAP)], gate_hbm.at[eid])
            pltpu.sync_copy(slotb, slot_hbm.at[eid])
            cntb[pl.ds(0, L)] = jnp.full((L,), cnt, jnp.int32)
            pltpu.sync_copy(cntb, cnt_hbm.at[eid])

        plsc.subcore_barrier()

        el = sid // TPE
        s0 = (sid % TPE) * RPT
        pltpu.sync_copy(sh_tok.at[el, pl.ds(s0, RPT)], idxb)
        r0 = (cid * EPC + el) * CAP + s0
        bufs = (rowa, rowb)
        gsems = (gsem0, gsem1)
        wsems = (wsem0, wsem1)
        gd = [None] * NGC
        wd = [None] * NGC
        gd[0] = pltpu.async_copy(x_hbm.at[idxb.at[pl.ds(0, GC)]], bufs[0],
                                 gsems[0])
        for k in range(NGC):
            b = k % 2
            if k + 1 < NGC:
                if k - 1 >= 0:
                    wd[k - 1].wait()
                gd[k + 1] = pltpu.async_copy(
                    x_hbm.at[idxb.at[pl.ds((k + 1) * GC, GC)]],
                    bufs[(k + 1) % 2], gsems[(k + 1) % 2])
            gd[k].wait()
            wd[k] = pltpu.async_copy(bufs[b],
                                     xe_hbm.at[pl.ds(r0 + k * GC, GC)],
                                     wsems[b])
        wd[NGC - 2].wait()
        wd[NGC - 1].wait()

    return dispatch


def _ffn_body(xe_ref, w1_ref, b1_ref, w2_ref, b2_ref, gate_ref, y_ref):
    xp = lax.bitcast_convert_type(xe_ref[...], jnp.uint32)
    lo = lax.bitcast_convert_type((xp & 0xFFFF).astype(jnp.uint16),
                                  jnp.bfloat16)
    hi = lax.bitcast_convert_type((xp >> 16).astype(jnp.uint16),
                                  jnp.bfloat16)
    xe = jnp.concatenate([lo, hi], axis=1)
    h = jnp.dot(xe, w1_ref[0].astype(jnp.bfloat16),
                preferred_element_type=jnp.float32)
    h = jnp.maximum(h + b1_ref[0], 0.0)
    part = jnp.dot(h.astype(jnp.bfloat16), w2_ref[0].astype(jnp.bfloat16),
                   preferred_element_type=jnp.float32)
    y_ref[...] = (part + b2_ref[0]) * gate_ref[...]


def _ffn_call(xe, W1, b1, W2, b2, gate_map, CAP):
    E, C, F = W1.shape
    return pl.pallas_call(
        _ffn_body,
        grid=(E,),
        in_specs=[
            pl.BlockSpec((CAP, C // 2), lambda e: (e, 0)),
            pl.BlockSpec((1, C, F), lambda e: (e, 0, 0)),
            pl.BlockSpec((1, 1, F), lambda e: (e, 0, 0)),
            pl.BlockSpec((1, F, C), lambda e: (e, 0, 0)),
            pl.BlockSpec((1, 1, C), lambda e: (e, 0, 0)),
            pl.BlockSpec((CAP, 1), lambda e: (e, 0)),
        ],
        out_specs=pl.BlockSpec((CAP, C), lambda e: (e, 0)),
        out_shape=jax.ShapeDtypeStruct((E * CAP, C), jnp.float32),
        name="expert_ffn",
        compiler_params=pltpu.CompilerParams(
            dimension_semantics=("arbitrary",),
            vmem_limit_bytes=110 * 1024 * 1024),
    )(xe, W1, b1.reshape(E, 1, F), W2, b2.reshape(E, 1, C),
      gate_map.reshape(E * CAP, 1))


def _make_combine_kernel(N, C, E, CAP):
    TPW = N // NW
    NCH = TPW // L
    GC = 32
    NGC = TPW // GC
    CV = C // L
    mesh = plsc.VectorSubcoreMesh(core_axis_name="c", subcore_axis_name="s")

    @functools.partial(
        pl.kernel,
        out_type=jax.ShapeDtypeStruct((N, C), jnp.float32),
        mesh=mesh,
        compiler_params=pltpu.CompilerParams(needs_layout_passes=False),
        scratch_types=[
            pltpu.VMEM((E, TPW), jnp.int32),
            pltpu.VMEM((E, L), jnp.int32),
            pltpu.VMEM((TPW,), jnp.int32),
            pltpu.VMEM((TPW,), jnp.int32),
            pltpu.VMEM((TPW,), jnp.int32),
            pltpu.VMEM((TPW,), jnp.int32),
            pltpu.VMEM((GC, C), jnp.float32),
            pltpu.VMEM((GC, C), jnp.float32),
            pltpu.VMEM((GC, C), jnp.float32),
            pltpu.VMEM((GC, C), jnp.float32),
            pltpu.SemaphoreType.DMA,
            pltpu.SemaphoreType.DMA,
            pltpu.SemaphoreType.DMA,
            pltpu.SemaphoreType.DMA,
            pltpu.SemaphoreType.DMA,
            pltpu.SemaphoreType.DMA,
        ],
    )
    def combine(y_hbm, slot_hbm, cnt_hbm, e1_hbm, e2_hbm, out_hbm,
                slotm, cnts, e1b, e2b, p1b, p2b,
                a1, a2, b1, b2, ga1, ga2, gb1, gb2, wsa, wsb):
        cid = lax.axis_index("c")
        sid = lax.axis_index("s")
        wid = sid * NC + cid
        t0 = wid * TPW

        pltpu.sync_copy(slot_hbm.at[:, pl.ds(t0, TPW)], slotm)
        pltpu.sync_copy(cnt_hbm, cnts)
        pltpu.sync_copy(e1_hbm.at[0, pl.ds(t0, TPW)], e1b)
        pltpu.sync_copy(e2_hbm.at[0, pl.ds(t0, TPW)], e2b)

        ez = jnp.int32(-1)
        for e in range(E):
            tot = cnts[e][0]
            take = jnp.logical_and(tot < CAP, ez < 0)
            ez = jnp.where(take, jnp.int32(e), ez)
        zero_flat = jnp.where(ez >= 0, ez * CAP + (CAP - 1), 0)

        for c in range(NCH):
            ve1 = e1b[pl.ds(c * L, L)]
            ve2 = e2b[pl.ds(c * L, L)]
            s1 = jnp.zeros((L,), jnp.int32)
            s2 = jnp.zeros((L,), jnp.int32)
            for e in range(E):
                row = slotm[e, pl.ds(c * L, L)]
                s1 = jnp.where(ve1 == e, row, s1)
                s2 = jnp.where(ve2 == e, row, s2)
            p1b[pl.ds(c * L, L)] = jnp.where(s1 < CAP, ve1 * CAP + s1,
                                             zero_flat)
            p2b[pl.ds(c * L, L)] = jnp.where(s2 < CAP, ve2 * CAP + s2,
                                             zero_flat)

        sets = ((a1, a2, ga1, ga2, wsa), (b1, b2, gb1, gb2, wsb))

        def fire(k):
            u1, u2, s1_, s2_, _ = sets[k % 2]
            d1 = pltpu.async_copy(y_hbm.at[p1b.at[pl.ds(k * GC, GC)]], u1,
                                  s1_)
            d2 = pltpu.async_copy(y_hbm.at[p2b.at[pl.ds(k * GC, GC)]], u2,
                                  s2_)
            return (d1, d2)

        gd = [None] * NGC
        wd = [None] * NGC
        gd[0] = fire(0)
        for k in range(NGC):
            u1, u2, _, _, ws = sets[k % 2]
            if k + 1 < NGC:
                if k - 1 >= 0:
                    wd[k - 1].wait()
                gd[k + 1] = fire(k + 1)
            gd[k][0].wait()
            gd[k][1].wait()

            def _add(j, carry):
                for v in range(CV):
                    u1[j, pl.ds(v * L, L)] = (u1[j, pl.ds(v * L, L)]
                                              + u2[j, pl.ds(v * L, L)])
                return carry

            lax.fori_loop(0, GC, _add, 0)
            wd[k] = pltpu.async_copy(u1, out_hbm.at[pl.ds(t0 + k * GC, GC)],
                                     ws)
        wd[NGC - 2].wait()
        wd[NGC - 1].wait()

    return combine


def kernel(x, noise, Wr, br, Wn, bn, W1, b1, W2, b2):
    Bb, Tt, C = x.shape
    N = Bb * Tt
    E = Wr.shape[1]
    CAP = (N * TOP_K) // E

    xf = x.reshape(N, C)
    noiseT = noise.reshape(N, E).T
    e1, e2, g1, g2, xbf = _router_call(
        xf, Wr.T, br.reshape(E, 1), Wn.T, bn.reshape(E, 1), noiseT)

    dispatch = _make_dispatch_kernel(N, C // 2, E, CAP)
    xe, gate_map, slot_mat, counts = dispatch(xbf, e1, e2, g1, g2)

    y = _ffn_call(xe, W1, b1, W2, b2, gate_map, CAP)

    combine = _make_combine_kernel(N, C, E, CAP)
    out = combine(y, slot_mat, counts, e1, e2)
    return out.reshape(Bb, Tt, C)
